# baseline (speedup 1.0000x reference)
import jax
import jax.numpy as jnp
from jax.experimental import pallas as pl
from jax.experimental.pallas import tpu as pltpu

B, L, D = 16, 2048, 300


def _body(lenp_ref, lenh_ref, p_ref, h_ref, out_ref):
    b = pl.program_id(0)
    lp = lenp_ref[b]
    lh = lenh_ref[b]
    iot = jax.lax.broadcasted_iota(jnp.int32, (L, 1), 0)
    maskp = (iot < lp).astype(jnp.float32)
    maskh = (iot < lh).astype(jnp.float32)
    p = jnp.sum(p_ref[0] * maskp, axis=0) / lp.astype(jnp.float32)
    h = jnp.sum(h_ref[0] * maskh, axis=0) / lh.astype(jnp.float32)
    out_ref[0, 0, :] = jnp.concatenate([p, h, jnp.abs(p - h), p * h], axis=0)


def kernel(premises, lengths_premises, hypothesis, lengths_hypothesis):
    out = pl.pallas_call(
        _body,
        grid=(B,),
        in_specs=[
            pl.BlockSpec(memory_space=pltpu.SMEM),
            pl.BlockSpec(memory_space=pltpu.SMEM),
            pl.BlockSpec((1, L, D), lambda b: (b, 0, 0)),
            pl.BlockSpec((1, L, D), lambda b: (b, 0, 0)),
        ],
        out_specs=pl.BlockSpec((1, 1, 4 * D), lambda b: (b, 0, 0)),
        out_shape=jax.ShapeDtypeStruct((B, 1, 4 * D), jnp.float32),
    )(lengths_premises, lengths_hypothesis, premises, hypothesis)
    return out.reshape(B, 4 * D)
